# Initial kernel scaffold; baseline (speedup 1.0000x reference)
import jax
import jax.numpy as jnp
from jax.experimental import pallas as pl


def kernel(x, edge_index, W1, b1, W2, b2):
    raise NotImplementedError("write your pallas kernel here")


# speedup vs baseline: 20.5498x; 20.5498x over previous
import functools

import jax
import jax.numpy as jnp
from jax import lax
from jax.experimental import pallas as pl
from jax.experimental.pallas import tpu as pltpu
from jax.experimental.pallas import tpu_sc as plsc

N = 10000
E = 320000
D_IN = 128
HC = 128
N_CLASSES = 16

NC = 2
NS = 16
NW = NC * NS
EPW = E // NW
NPAD = 10240


def _sc_agg(width, gather, ch, n_tables=1):
    nch = -(-EPW // ch)
    rpt = NPAD // NS
    rch = rpt // ch
    mesh = plsc.VectorSubcoreMesh(core_axis_name="c", subcore_axis_name="s")
    scratch = []
    if gather:
        scratch.append(pltpu.VMEM((nch, ch), jnp.int32))
    scratch += [
        pltpu.VMEM((nch, ch), jnp.int32),
        pltpu.VMEM((2, ch, width), jnp.float32),
        pltpu.VMEM_SHARED((NPAD, width), jnp.float32),
        pltpu.SemaphoreType.DMA((2,)),
        pltpu.SemaphoreType.DMA((2,)),
    ]

    def body(*refs):
        if gather:
            src_hbm, dst_hbm = refs[0], refs[1]
            y_hbms = refs[2:2 + n_tables]
            out_hbms = refs[2 + n_tables:2 + 2 * n_tables]
            idx_s, idx_d, rows, acc, gsem, ssem = refs[2 + 2 * n_tables:]
        else:
            dst_hbm = refs[0]
            out_hbms = refs[1:1 + n_tables]
            idx_d, rows, acc, gsem, ssem = refs[1 + n_tables:]
        cid = lax.axis_index("c")
        sid = lax.axis_index("s")
        wid = cid * NS + sid

        z16 = jnp.zeros((16,), jnp.float32)
        o16 = jnp.ones((16,), jnp.float32)

        if gather:
            pltpu.sync_copy(src_hbm.at[wid], idx_s)
        pltpu.sync_copy(dst_hbm.at[wid], idx_d)

        for y_hbm, out_hbm in zip(y_hbms if gather else [None] * n_tables,
                                  out_hbms):
            def fill_row(i, _):
                for j in range(width // 16):
                    rows[0, i, pl.ds(16 * j, 16)] = z16
                    if not gather:
                        rows[1, i, pl.ds(16 * j, 16)] = o16
                return 0

            lax.fori_loop(0, ch, fill_row, 0)

            for k in range(rch):
                pltpu.sync_copy(rows.at[0],
                                acc.at[pl.ds(sid * rpt + k * ch, ch)])
            plsc.subcore_barrier()

            if gather:
                def start_g(c, b):
                    pltpu.async_copy(y_hbm.at[idx_s.at[c]], rows.at[b],
                                     gsem.at[b])

                def step(c, b):
                    pltpu.make_async_copy(
                        y_hbm.at[idx_s.at[c]], rows.at[b], gsem.at[b]).wait()
                    pltpu.sync_copy(rows.at[b], acc.at[idx_d.at[c]], add=True)

                    @pl.when(c + 2 < nch)
                    def _():
                        start_g(c + 2, b)

                start_g(0, 0)
                start_g(1, 1)

                def pair(k, _):
                    step(2 * k, 0)
                    step(2 * k + 1, 1)
                    return 0

                lax.fori_loop(0, nch // 2, pair, 0)
                if nch % 2:
                    step(nch - 1, 0)
            else:
                def dstart(c, b):
                    pltpu.async_copy(rows.at[1], acc.at[idx_d.at[c]],
                                     ssem.at[b], add=True)

                def dwait(c, b):
                    pltpu.make_async_copy(rows.at[1], acc.at[idx_d.at[c]],
                                          ssem.at[b]).wait()

                def dpair(k, _):
                    dstart(2 * k, 0)
                    dstart(2 * k + 1, 1)
                    dwait(2 * k, 0)
                    dwait(2 * k + 1, 1)
                    return 0

                lax.fori_loop(0, nch // 2, dpair, 0)
                if nch % 2:
                    dstart(nch - 1, 0)
                    dwait(nch - 1, 0)

            plsc.subcore_barrier()

            for k in range(rch):
                r0 = sid * rpt + k * ch
                pltpu.sync_copy(acc.at[pl.ds(r0, ch)],
                                out_hbm.at[cid].at[pl.ds(r0, ch)])

    out_t = [jax.ShapeDtypeStruct((NC, NPAD, width), jnp.float32)
             ] * n_tables
    return pl.kernel(
        body,
        out_type=out_t if n_tables > 1 else out_t[0],
        mesh=mesh,
        scratch_types=scratch,
        compiler_params=pltpu.CompilerParams(use_tc_tiling_on_sc=False),
    )


def _mm1_body(x_ref, w_ref, o_ref):
    o_ref[...] = jnp.dot(x_ref[...], w_ref[...],
                         preferred_element_type=jnp.float32)


def _prep_body(p0_ref, p1_ref, xw_ref, ya_ref, yb_ref, dinv_ref):
    deg = 1.0 + p0_ref[:, 0:1] + p1_ref[:, 0:1]
    dinv = lax.rsqrt(deg)
    y = xw_ref[...] * dinv
    ya_ref[...] = y[:, :HC // 2]
    yb_ref[...] = y[:, HC // 2:]
    dinv_ref[...] = jnp.broadcast_to(dinv, (N, 16))


def _mid_body(aa0_ref, aa1_ref, ab0_ref, ab1_ref, ya_ref, yb_ref,
              dinv_ref, w2_ref, b1_ref, z_ref):
    dinv = dinv_ref[:, 0:1]
    agg = jnp.concatenate(
        [aa0_ref[...] + aa1_ref[...] + ya_ref[...],
         ab0_ref[...] + ab1_ref[...] + yb_ref[...]], axis=1)
    h = dinv * agg + b1_ref[...]
    g = jnp.dot(h, w2_ref[...], preferred_element_type=jnp.float32)
    z_ref[...] = g * dinv


def _final_body(a0_ref, a1_ref, z_ref, dinv_ref, b2_ref, o_ref):
    dinv = dinv_ref[:, 0:1]
    o_ref[...] = dinv * (a0_ref[...] + a1_ref[...] + z_ref[...]) + b2_ref[...]


def _f32(shape):
    return jax.ShapeDtypeStruct(shape, jnp.float32)


def _chunked(idx, ch, pad_value):
    nch = -(-EPW // ch)
    pad = nch * ch - EPW
    return jnp.pad(idx.reshape(NW, EPW), ((0, 0), (0, pad)),
                   constant_values=pad_value).reshape(NW, nch, ch)


def kernel(x, edge_index, W1, b1, W2, b2):
    ei = edge_index.astype(jnp.int32)
    src = _chunked(ei[0], 128, 0)
    dst = _chunked(ei[1], 128, N)

    degp = _sc_agg(16, gather=False, ch=128)(dst)
    xw = pl.pallas_call(_mm1_body, out_shape=_f32((N, HC)))(x, W1)
    ya, yb, dinv = pl.pallas_call(
        _prep_body,
        out_shape=[_f32((N, HC // 2)), _f32((N, HC // 2)), _f32((N, 16))],
    )(degp[0, :N], degp[1, :N], xw)

    a1a, a1b = _sc_agg(HC // 2, gather=True, ch=128, n_tables=2)(
        src, dst, ya, yb)
    z = pl.pallas_call(
        _mid_body, out_shape=_f32((N, N_CLASSES)),
    )(a1a[0, :N], a1a[1, :N], a1b[0, :N], a1b[1, :N], ya, yb,
      dinv, W2, b1.reshape(1, HC))

    agg2 = _sc_agg(N_CLASSES, gather=True, ch=128)(src, dst, z)
    out = pl.pallas_call(
        _final_body, out_shape=_f32((N, N_CLASSES)),
    )(agg2[0, :N], agg2[1, :N], z, dinv, b2.reshape(1, N_CLASSES))
    return out


# speedup vs baseline: 22.2630x; 1.0834x over previous
import functools

import jax
import jax.numpy as jnp
from jax import lax
from jax.experimental import pallas as pl
from jax.experimental.pallas import tpu as pltpu
from jax.experimental.pallas import tpu_sc as plsc

N = 10000
E = 320000
D_IN = 128
HC = 128
N_CLASSES = 16

NC = 2
NS = 16
NW = NC * NS
EPW = E // NW
NPAD = 10240


def _sc_agg(width, gather, ch, n_tables=1):
    nch = -(-EPW // ch)
    rpt = NPAD // NS
    rch = rpt // ch
    mesh = plsc.VectorSubcoreMesh(core_axis_name="c", subcore_axis_name="s")
    nb = 4
    scratch = []
    if gather:
        scratch.append(pltpu.VMEM((nch, ch), jnp.int32))
    scratch += [
        pltpu.VMEM((nch, ch), jnp.int32),
        pltpu.VMEM((nb if gather else 2, ch, width), jnp.float32),
        pltpu.VMEM_SHARED((NPAD, width), jnp.float32),
        pltpu.SemaphoreType.DMA((nb,)),
        pltpu.SemaphoreType.DMA((nb,)),
    ]

    def body(*refs):
        if gather:
            src_hbm, dst_hbm = refs[0], refs[1]
            y_hbms = refs[2:2 + n_tables]
            out_hbms = refs[2 + n_tables:2 + 2 * n_tables]
            idx_s, idx_d, rows, acc, gsem, ssem = refs[2 + 2 * n_tables:]
        else:
            dst_hbm = refs[0]
            out_hbms = refs[1:1 + n_tables]
            idx_d, rows, acc, gsem, ssem = refs[1 + n_tables:]
        cid = lax.axis_index("c")
        sid = lax.axis_index("s")
        wid = cid * NS + sid

        z16 = jnp.zeros((16,), jnp.float32)
        o16 = jnp.ones((16,), jnp.float32)

        if gather:
            pltpu.sync_copy(src_hbm.at[wid], idx_s)
        pltpu.sync_copy(dst_hbm.at[wid], idx_d)

        for y_hbm, out_hbm in zip(y_hbms if gather else [None] * n_tables,
                                  out_hbms):
            def fill_row(i, _):
                for j in range(width // 16):
                    rows[0, i, pl.ds(16 * j, 16)] = z16
                    if not gather:
                        rows[1, i, pl.ds(16 * j, 16)] = o16
                return 0

            lax.fori_loop(0, ch, fill_row, 0)

            for k in range(rch):
                pltpu.sync_copy(rows.at[0],
                                acc.at[pl.ds(sid * rpt + k * ch, ch)])
            plsc.subcore_barrier()

            if gather:
                def start_g(c, b):
                    pltpu.async_copy(y_hbm.at[idx_s.at[c]], rows.at[b],
                                     gsem.at[b])

                def wait_g(c, b):
                    pltpu.make_async_copy(
                        y_hbm.at[idx_s.at[c]], rows.at[b], gsem.at[b]).wait()

                def start_s(c, b):
                    pltpu.async_copy(rows.at[b], acc.at[idx_d.at[c]],
                                     ssem.at[b], add=True)

                def wait_s(b):
                    pltpu.make_async_copy(rows.at[b], acc.at[idx_d.at[0]],
                                          ssem.at[b]).wait()

                start_g(0, 0)
                start_g(1, 1)
                start_g(2, 2)
                wait_g(0, 0)
                start_s(0, 0)
                start_g(3, 3)

                ngrp = (nch - 3) // 4

                def group(k, _):
                    c0 = 4 * k + 1
                    for j in range(4):
                        c = c0 + j
                        b = (1 + j) % nb
                        wait_g(c, b)
                        start_s(c, b)
                        bn = (b + 3) % nb

                        @pl.when(c + 3 < nch)
                        def _():
                            wait_s(bn)
                            start_g(c + 3, bn)
                    return 0

                lax.fori_loop(0, ngrp, group, 0)
                for c in range(4 * ngrp + 1, nch):
                    b = c % nb
                    wait_g(c, b)
                    if c + 3 < nch:
                        wait_s((b + 3) % nb)
                        start_g(c + 3, (b + 3) % nb)
                    start_s(c, b)
                for c in range(max(nch - 4, 0), nch):
                    wait_s(c % nb)
            else:
                def dstart(c, j):
                    pltpu.async_copy(rows.at[1], acc.at[idx_d.at[c]],
                                     ssem.at[j], add=True)

                def dwait(c, j):
                    pltpu.make_async_copy(rows.at[1], acc.at[idx_d.at[c]],
                                          ssem.at[j]).wait()

                def dgroup(k, _):
                    for j in range(4):
                        dstart(4 * k + j, j)
                    for j in range(4):
                        dwait(4 * k + j, j)
                    return 0

                lax.fori_loop(0, nch // 4, dgroup, 0)
                for j in range(nch % 4):
                    dstart(4 * (nch // 4) + j, j)
                for j in range(nch % 4):
                    dwait(4 * (nch // 4) + j, j)

            plsc.subcore_barrier()

            for k in range(rch):
                r0 = sid * rpt + k * ch
                pltpu.sync_copy(acc.at[pl.ds(r0, ch)],
                                out_hbm.at[cid].at[pl.ds(r0, ch)])

    out_t = [jax.ShapeDtypeStruct((NC, NPAD, width), jnp.float32)
             ] * n_tables
    return pl.kernel(
        body,
        out_type=out_t if n_tables > 1 else out_t[0],
        mesh=mesh,
        scratch_types=scratch,
        compiler_params=pltpu.CompilerParams(use_tc_tiling_on_sc=False),
    )


def _prep_body(p0_ref, p1_ref, x_ref, w1_ref, ya_ref, yb_ref, dinv_ref):
    xw = jnp.dot(x_ref[...], w1_ref[...], preferred_element_type=jnp.float32)
    deg = 1.0 + p0_ref[:, 0:1] + p1_ref[:, 0:1]
    dinv = lax.rsqrt(deg)
    y = xw * dinv
    ya_ref[...] = y[:, :HC // 2]
    yb_ref[...] = y[:, HC // 2:]
    dinv_ref[...] = jnp.broadcast_to(dinv, (N, 16))


def _mid_body(aa0_ref, aa1_ref, ab0_ref, ab1_ref, ya_ref, yb_ref,
              dinv_ref, w2_ref, b1_ref, z_ref):
    dinv = dinv_ref[:, 0:1]
    agg = jnp.concatenate(
        [aa0_ref[...] + aa1_ref[...] + ya_ref[...],
         ab0_ref[...] + ab1_ref[...] + yb_ref[...]], axis=1)
    h = dinv * agg + b1_ref[...]
    g = jnp.dot(h, w2_ref[...], preferred_element_type=jnp.float32)
    z_ref[...] = g * dinv


def _final_body(a0_ref, a1_ref, z_ref, dinv_ref, b2_ref, o_ref):
    dinv = dinv_ref[:, 0:1]
    o_ref[...] = dinv * (a0_ref[...] + a1_ref[...] + z_ref[...]) + b2_ref[...]


def _f32(shape):
    return jax.ShapeDtypeStruct(shape, jnp.float32)


def _chunked(idx, ch, pad_value):
    nch = -(-EPW // ch)
    pad = nch * ch - EPW
    return jnp.pad(idx.reshape(NW, EPW), ((0, 0), (0, pad)),
                   constant_values=pad_value).reshape(NW, nch, ch)


def kernel(x, edge_index, W1, b1, W2, b2):
    ei = edge_index.astype(jnp.int32)
    src = _chunked(ei[0], 128, 0)
    dst = _chunked(ei[1], 128, N)

    degp = _sc_agg(16, gather=False, ch=128)(dst)
    ya, yb, dinv = pl.pallas_call(
        _prep_body,
        out_shape=[_f32((N, HC // 2)), _f32((N, HC // 2)), _f32((N, 16))],
    )(degp[0, :N], degp[1, :N], x, W1)

    a1a, a1b = _sc_agg(HC // 2, gather=True, ch=128, n_tables=2)(
        src, dst, ya, yb)
    z = pl.pallas_call(
        _mid_body, out_shape=_f32((N, N_CLASSES)),
    )(a1a[0, :N], a1a[1, :N], a1b[0, :N], a1b[1, :N], ya, yb,
      dinv, W2, b1.reshape(1, HC))

    agg2 = _sc_agg(N_CLASSES, gather=True, ch=128)(src, dst, z)
    out = pl.pallas_call(
        _final_body, out_shape=_f32((N, N_CLASSES)),
    )(agg2[0, :N], agg2[1, :N], z, dinv, b2.reshape(1, N_CLASSES))
    return out


# speedup vs baseline: 37.6887x; 1.6929x over previous
import functools

import jax
import jax.numpy as jnp
from jax import lax
from jax.experimental import pallas as pl
from jax.experimental.pallas import tpu as pltpu
from jax.experimental.pallas import tpu_sc as plsc

N = 10000
E = 320000
D_IN = 128
HC = 128
N_CLASSES = 16

NC = 2
NS = 16
NW = NC * NS
CH = 128
NCHT = E // CH
NCHW = NCHT // NW
NEXTRA = NCHT - NCHW * NW
NPAD = 10240
RPT = NPAD // NS
RCH = RPT // CH
NB = 4


def _sc_agg(width, gather, n_tables=1):
    mesh = plsc.VectorSubcoreMesh(core_axis_name="c", subcore_axis_name="s")
    nbuf = NB if gather else 2
    scratch = []
    if gather:
        scratch.append(pltpu.VMEM((NCHW + 1, CH), jnp.int32))
    scratch += [
        pltpu.VMEM((NCHW + 1, CH), jnp.int32),
        pltpu.VMEM((nbuf, CH, width), jnp.float32),
        pltpu.VMEM_SHARED((NPAD, width), jnp.float32),
        pltpu.SemaphoreType.DMA((NB,)),
        pltpu.SemaphoreType.DMA((NB,)),
    ]

    def body(*refs):
        if gather:
            ei_hbm = refs[0]
            y_hbms = refs[1:1 + n_tables]
            out_hbms = refs[1 + n_tables:1 + 2 * n_tables]
            idx_s, idx_d, rows, acc, gsem, ssem = refs[1 + 2 * n_tables:]
        else:
            ei_hbm = refs[0]
            out_hbms = refs[1:1 + n_tables]
            idx_d, rows, acc, gsem, ssem = refs[1 + n_tables:]
        cid = lax.axis_index("c")
        sid = lax.axis_index("s")
        wid = cid * NS + sid
        has_extra = wid < NEXTRA

        z16 = jnp.zeros((16,), jnp.float32)
        o16 = jnp.ones((16,), jnp.float32)

        if gather:
            pltpu.sync_copy(ei_hbm.at[0, pl.ds(wid * NCHW, NCHW)],
                            idx_s.at[pl.ds(0, NCHW)])
        pltpu.sync_copy(ei_hbm.at[1, pl.ds(wid * NCHW, NCHW)],
                        idx_d.at[pl.ds(0, NCHW)])

        @pl.when(has_extra)
        def _():
            if gather:
                pltpu.sync_copy(ei_hbm.at[0, pl.ds(NCHW * NW + wid, 1)],
                                idx_s.at[pl.ds(NCHW, 1)])
            pltpu.sync_copy(ei_hbm.at[1, pl.ds(NCHW * NW + wid, 1)],
                            idx_d.at[pl.ds(NCHW, 1)])

        for y_hbm, out_hbm in zip(y_hbms if gather else [None] * n_tables,
                                  out_hbms):
            def fill_row(i, _):
                for j in range(width // 16):
                    rows[0, i, pl.ds(16 * j, 16)] = z16
                    if not gather:
                        rows[1, i, pl.ds(16 * j, 16)] = o16
                return 0

            lax.fori_loop(0, CH, fill_row, 0)

            for k in range(RCH):
                pltpu.sync_copy(rows.at[0],
                                acc.at[pl.ds(sid * RPT + k * CH, CH)])
            plsc.subcore_barrier()

            if gather:
                def start_g(c, b):
                    pltpu.async_copy(y_hbm.at[idx_s.at[c]], rows.at[b],
                                     gsem.at[b])

                def wait_g(c, b):
                    pltpu.make_async_copy(
                        y_hbm.at[idx_s.at[c]], rows.at[b], gsem.at[b]).wait()

                def start_s(c, b):
                    pltpu.async_copy(rows.at[b], acc.at[idx_d.at[c]],
                                     ssem.at[b], add=True)

                def wait_s(b):
                    pltpu.make_async_copy(rows.at[b], acc.at[idx_d.at[0]],
                                          ssem.at[b]).wait()

                start_g(0, 0)
                start_g(1, 1)
                start_g(2, 2)
                wait_g(0, 0)
                start_s(0, 0)
                start_g(3, 3)

                ngrp = (NCHW - 3) // 4

                def group(k, _):
                    c0 = 4 * k + 1
                    for j in range(4):
                        c = c0 + j
                        b = (1 + j) % NB
                        wait_g(c, b)
                        start_s(c, b)
                        bn = (b + 3) % NB

                        @pl.when(c + 3 < NCHW)
                        def _():
                            wait_s(bn)
                            start_g(c + 3, bn)
                    return 0

                lax.fori_loop(0, ngrp, group, 0)
                for c in range(4 * ngrp + 1, NCHW):
                    b = c % NB
                    wait_g(c, b)
                    if c + 3 < NCHW:
                        wait_s((b + 3) % NB)
                        start_g(c + 3, (b + 3) % NB)
                    start_s(c, b)
                for c in range(max(NCHW - 4, 0), NCHW):
                    wait_s(c % NB)

                @pl.when(has_extra)
                def _():
                    pltpu.sync_copy(y_hbm.at[idx_s.at[NCHW]], rows.at[0])
                    pltpu.sync_copy(rows.at[0], acc.at[idx_d.at[NCHW]],
                                    add=True)
            else:
                def dstart(c, j):
                    pltpu.async_copy(rows.at[1], acc.at[idx_d.at[c]],
                                     ssem.at[j], add=True)

                def dwait(c, j):
                    pltpu.make_async_copy(rows.at[1], acc.at[idx_d.at[c]],
                                          ssem.at[j]).wait()

                def dgroup(k, _):
                    for j in range(4):
                        dstart(4 * k + j, j)
                    for j in range(4):
                        dwait(4 * k + j, j)
                    return 0

                lax.fori_loop(0, NCHW // 4, dgroup, 0)
                for j in range(NCHW % 4):
                    dstart(4 * (NCHW // 4) + j, j)
                for j in range(NCHW % 4):
                    dwait(4 * (NCHW // 4) + j, j)

                @pl.when(has_extra)
                def _():
                    pltpu.sync_copy(rows.at[1], acc.at[idx_d.at[NCHW]],
                                    add=True)

            plsc.subcore_barrier()

            for k in range(RCH):
                r0 = sid * RPT + k * CH
                pltpu.sync_copy(acc.at[pl.ds(r0, CH)],
                                out_hbm.at[cid].at[pl.ds(r0, CH)])

    out_t = [jax.ShapeDtypeStruct((NC, NPAD, width), jnp.float32)
             ] * n_tables
    return pl.kernel(
        body,
        out_type=out_t if n_tables > 1 else out_t[0],
        mesh=mesh,
        scratch_types=scratch,
        compiler_params=pltpu.CompilerParams(use_tc_tiling_on_sc=False),
    )


def _prep_body(degp_ref, x_ref, w1_ref, ya_ref, yb_ref, dinv_ref):
    xw = jnp.dot(x_ref[...], w1_ref[...], preferred_element_type=jnp.float32)
    deg = 1.0 + degp_ref[0, :N, 0:1] + degp_ref[1, :N, 0:1]
    dinv = lax.rsqrt(deg)
    y = xw * dinv
    ya_ref[...] = y[:, :HC // 2]
    yb_ref[...] = y[:, HC // 2:]
    dinv_ref[...] = jnp.broadcast_to(dinv, (N, 16))


def _mid_body(aa_ref, ab_ref, ya_ref, yb_ref, dinv_ref, w2_ref, b1_ref,
              z_ref):
    dinv = dinv_ref[:, 0:1]
    agg = jnp.concatenate(
        [aa_ref[0, :N, :] + aa_ref[1, :N, :] + ya_ref[...],
         ab_ref[0, :N, :] + ab_ref[1, :N, :] + yb_ref[...]], axis=1)
    h = dinv * agg + b1_ref[...]
    g = jnp.dot(h, w2_ref[...], preferred_element_type=jnp.float32)
    z_ref[...] = g * dinv


def _final_body(a2_ref, z_ref, dinv_ref, b2_ref, o_ref):
    dinv = dinv_ref[:, 0:1]
    o_ref[...] = dinv * (a2_ref[0, :N, :] + a2_ref[1, :N, :] + z_ref[...]) \
        + b2_ref[...]


def _f32(shape):
    return jax.ShapeDtypeStruct(shape, jnp.float32)


def kernel(x, edge_index, W1, b1, W2, b2):
    ei = edge_index.astype(jnp.int32).reshape(2, NCHT, CH)

    degp = _sc_agg(16, gather=False)(ei)
    ya, yb, dinv = pl.pallas_call(
        _prep_body,
        out_shape=[_f32((N, HC // 2)), _f32((N, HC // 2)), _f32((N, 16))],
    )(degp, x, W1)

    a1a, a1b = _sc_agg(HC // 2, gather=True, n_tables=2)(
        ei, ya, yb)
    z = pl.pallas_call(
        _mid_body, out_shape=_f32((N, N_CLASSES)),
    )(a1a, a1b, ya, yb, dinv, W2, b1.reshape(1, HC))

    agg2 = _sc_agg(N_CLASSES, gather=True)(ei, z)
    out = pl.pallas_call(
        _final_body, out_shape=_f32((N, N_CLASSES)),
    )(agg2, z, dinv, b2.reshape(1, N_CLASSES))
    return out
